# Initial kernel scaffold; baseline (speedup 1.0000x reference)
import jax
import jax.numpy as jnp
from jax.experimental import pallas as pl


def kernel(node_features, edge_index, text_embeddings, W_proj, b_proj, Wg, W1, b1, W2, b2):
    raise NotImplementedError("write your pallas kernel here")


# speedup vs baseline: 13.6229x; 13.6229x over previous
import jax
import jax.numpy as jnp
from jax import lax
from jax.experimental import pallas as pl
from jax.experimental.pallas import tpu as pltpu
from jax.experimental.pallas import tpu_sc as plsc

_N = 10000
_E = 320000
_H = 128
_HH = 64
_TD = 384
_NL = 3
_NS = 16
_NPAD = 10240
_NZ = _NPAD // _NS
_EPT = _E // _NS
_EPT2 = _E // (2 * _NS)
_DW = 800
_W = 200
_ZCH = 160
_RB = 1000

_f32 = jnp.float32


def _pool_body(te_ref, o_ref):
    o_ref[...] = jnp.mean(te_ref[...], axis=0, keepdims=True)


def _hyper_body(p_ref, wg_ref, o_ref):
    o_ref[0] = jnp.dot(p_ref[...], wg_ref[0],
                       preferred_element_type=_f32)


def _layer0_body(nf_ref, wp_ref, bp_ref, dego_ref, w_ref, o_ref):
    x = jnp.dot(nf_ref[...], wp_ref[...], preferred_element_type=_f32)
    x = x + bp_ref[...]
    s = lax.rsqrt(jnp.maximum(dego_ref[...], 1.0))
    o_ref[...] = jnp.dot(x * s, w_ref[...], preferred_element_type=_f32)


def _layermid_body(agg_ref, dego_ref, degi_ref, w_ref, o_ref):
    x = agg_ref[0] + agg_ref[1]
    t = lax.rsqrt(jnp.maximum(degi_ref[...], 1.0))
    s = lax.rsqrt(jnp.maximum(dego_ref[...], 1.0))
    x = jnp.maximum(x * t, 0.0) * s
    o_ref[...] = jnp.dot(x, w_ref[...], preferred_element_type=_f32)


def _head_body(agg_ref, degi_ref, w1_ref, b1_ref, w2_ref, b2_ref, o_ref):
    x = agg_ref[0] + agg_ref[1]
    t = lax.rsqrt(jnp.maximum(degi_ref[...], 1.0))
    x = x * t
    h = jnp.dot(x, w1_ref[...], preferred_element_type=_f32) + b1_ref[...]
    h = jnp.maximum(h, 0.0)
    o_ref[...] = jnp.dot(h, w2_ref[...], preferred_element_type=_f32) + b2_ref[...]


def _pool_call(te):
    return pl.pallas_call(
        _pool_body,
        out_shape=jax.ShapeDtypeStruct((1, _TD), _f32),
    )(te)


def _hyper_call(pooled, Wg):
    return pl.pallas_call(
        _hyper_body,
        grid=(_NL, 8),
        in_specs=[
            pl.BlockSpec((1, _TD), lambda l, j: (0, 0)),
            pl.BlockSpec((1, _TD, 2048), lambda l, j: (l, 0, j)),
        ],
        out_specs=pl.BlockSpec((1, 1, 2048), lambda l, j: (l, 0, j)),
        out_shape=jax.ShapeDtypeStruct((_NL, 1, _H * _H), _f32),
    )(pooled, Wg)


def _layer0_call(nf, Wp, bp, dego, W0):
    return pl.pallas_call(
        _layer0_body,
        grid=(_N // _RB,),
        in_specs=[
            pl.BlockSpec((_RB, _H), lambda i: (i, 0)),
            pl.BlockSpec((_H, _H), lambda i: (0, 0)),
            pl.BlockSpec((1, _H), lambda i: (0, 0)),
            pl.BlockSpec((_RB, 1), lambda i: (i, 0)),
            pl.BlockSpec((_H, _H), lambda i: (0, 0)),
        ],
        out_specs=pl.BlockSpec((_RB, _H), lambda i: (i, 0)),
        out_shape=jax.ShapeDtypeStruct((_N, _H), _f32),
    )(nf, Wp, bp, dego, W0)


def _layermid_call(agg, dego, degi, Wl):
    return pl.pallas_call(
        _layermid_body,
        grid=(_N // _RB,),
        in_specs=[
            pl.BlockSpec((2, _RB, _H), lambda i: (0, i, 0)),
            pl.BlockSpec((_RB, 1), lambda i: (i, 0)),
            pl.BlockSpec((_RB, 1), lambda i: (i, 0)),
            pl.BlockSpec((_H, _H), lambda i: (0, 0)),
        ],
        out_specs=pl.BlockSpec((_RB, _H), lambda i: (i, 0)),
        out_shape=jax.ShapeDtypeStruct((_N, _H), _f32),
    )(agg, dego, degi, Wl)


def _head_call(agg, degi, W1, b1, W2, b2):
    return pl.pallas_call(
        _head_body,
        grid=(_N // _RB,),
        in_specs=[
            pl.BlockSpec((2, _RB, _H), lambda i: (0, i, 0)),
            pl.BlockSpec((_RB, 1), lambda i: (i, 0)),
            pl.BlockSpec((_H, _HH), lambda i: (0, 0)),
            pl.BlockSpec((1, _HH), lambda i: (0, 0)),
            pl.BlockSpec((_HH, 1), lambda i: (0, 0)),
            pl.BlockSpec((1, 1), lambda i: (0, 0)),
        ],
        out_specs=pl.BlockSpec((_RB, 1), lambda i: (i, 0)),
        out_shape=jax.ShapeDtypeStruct((_N, 1), _f32),
    )(agg, degi, W1, b1, W2, b2)


def _vector_mesh():
    return plsc.VectorSubcoreMesh(
        core_axis_name="core", subcore_axis_name="subcore")


def _deg_call(src, dst):
    @pl.kernel(
        out_type=[jax.ShapeDtypeStruct((_NPAD,), _f32),
                  jax.ShapeDtypeStruct((_NPAD,), _f32)],
        mesh=_vector_mesh(),
        scratch_types=[
            pltpu.VMEM_SHARED((_NPAD,), _f32),
            pltpu.VMEM((1, _DW), jnp.int32),
            pltpu.VMEM((1, _DW), _f32),
        ],
    )
    def deg_kernel(src_hbm, dst_hbm, out0_hbm, out1_hbm, acc_sh, idx_v, val_v):
        c = lax.axis_index("core")
        s = lax.axis_index("subcore")

        @pl.loop(0, _DW // 16)
        def _zero(i):
            val_v[0, pl.ds(i * 16, 16)] = jnp.zeros((16,), _f32)

        pltpu.sync_copy(val_v.at[0, pl.ds(0, _NZ)],
                        acc_sh.at[pl.ds(s * _NZ, _NZ)])
        plsc.subcore_barrier()

        @pl.loop(0, _DW // 16)
        def _ones(i):
            val_v[0, pl.ds(i * 16, 16)] = jnp.ones((16,), _f32)

        @pl.loop(0, _EPT // _DW)
        def _win(k):
            off = s * _EPT + k * _DW

            @pl.when(c == 0)
            def _():
                pltpu.sync_copy(src_hbm.at[pl.ds(off, _DW)], idx_v.at[0])

            @pl.when(c == 1)
            def _():
                pltpu.sync_copy(dst_hbm.at[pl.ds(off, _DW)], idx_v.at[0])

            pltpu.sync_copy(val_v.at[0], acc_sh.at[idx_v.at[0]], add=True)

        plsc.subcore_barrier()

        @pl.when(c == 0)
        def _():
            pltpu.sync_copy(acc_sh.at[pl.ds(s * _NZ, _NZ)],
                            out0_hbm.at[pl.ds(s * _NZ, _NZ)])

        @pl.when(c == 1)
        def _():
            pltpu.sync_copy(acc_sh.at[pl.ds(s * _NZ, _NZ)],
                            out1_hbm.at[pl.ds(s * _NZ, _NZ)])

    return deg_kernel(src, dst)


def _agg_call(y, src, dst):
    @pl.kernel(
        out_type=jax.ShapeDtypeStruct((2, _NPAD, _H), _f32),
        mesh=_vector_mesh(),
        scratch_types=[
            pltpu.VMEM_SHARED((_NPAD, _H), _f32),
            pltpu.VMEM((1, _W), jnp.int32),
            pltpu.VMEM((1, _W), jnp.int32),
            pltpu.VMEM((_W, _H), _f32),
        ],
    )
    def agg_kernel(y_hbm, src_hbm, dst_hbm, out_hbm, acc_sh, sidx, didx,
                   rows_v):
        c = lax.axis_index("core")
        s = lax.axis_index("subcore")

        @pl.loop(0, _W)
        def _zrow(i):
            @pl.loop(0, _H // 16)
            def _zcol(j):
                rows_v[i, pl.ds(j * 16, 16)] = jnp.zeros((16,), _f32)

        @pl.loop(0, _NZ // _ZCH)
        def _zacc(i):
            pltpu.sync_copy(
                rows_v.at[pl.ds(0, _ZCH)],
                acc_sh.at[pl.ds(s * _NZ + i * _ZCH, _ZCH)])

        plsc.subcore_barrier()

        @pl.loop(0, _EPT2 // _W)
        def _win(k):
            off = (c * _NS + s) * _EPT2 + k * _W
            pltpu.sync_copy(src_hbm.at[pl.ds(off, _W)], sidx.at[0])
            pltpu.sync_copy(dst_hbm.at[pl.ds(off, _W)], didx.at[0])
            pltpu.sync_copy(y_hbm.at[sidx.at[0]], rows_v)
            pltpu.sync_copy(rows_v, acc_sh.at[didx.at[0]], add=True)

        plsc.subcore_barrier()
        pltpu.sync_copy(acc_sh.at[pl.ds(s * _NZ, _NZ)],
                        out_hbm.at[c, pl.ds(s * _NZ, _NZ)])

    return agg_kernel(y, src, dst)


def kernel(node_features, edge_index, text_embeddings, W_proj, b_proj,
           Wg, W1, b1, W2, b2):
    src = edge_index[0]
    dst = edge_index[1]
    deg_o, deg_i = _deg_call(src, dst)
    dego = deg_o[:_N].reshape(_N, 1)
    degi = deg_i[:_N].reshape(_N, 1)

    pooled = _pool_call(text_embeddings)
    W_all = _hyper_call(pooled, Wg)
    W_all = W_all.reshape(_NL, _H, _H)

    y = _layer0_call(node_features, W_proj, b_proj.reshape(1, _H),
                     dego, W_all[0])
    for l in range(1, _NL):
        agg = _agg_call(y, src, dst)
        y = _layermid_call(agg[:, :_N], dego, degi, W_all[l])
    agg = _agg_call(y, src, dst)

    return _head_call(agg[:, :_N], degi, W1, b1.reshape(1, _HH),
                      W2, b2.reshape(1, 1))
